# baseline (speedup 1.0000x reference)
import jax
import jax.numpy as jnp
from jax import lax
from jax.experimental import pallas as pl
from jax.experimental.pallas import tpu as pltpu
from jax.experimental.pallas import tpu_sc as plsc

_L = 16
_NC = 2
_NS = 16
_NW = _NC * _NS
_ENT = 10000


def _build(E, ENT, D, RB):
    D2 = 2 * D
    NKD = D // _L
    NK2 = D2 // _L
    NG = RB // _L
    NPASS = 3
    NBUF = 3
    NSEG = (ENT // _NW + 1 + NPASS - 1) // NPASS
    NSEGP = ((NSEG + 7) // 8) * 8

    def body(tid_hbm, rel_hbm, text_hbm, uwb_hbm, rs_hbm, out_hbm,
             relb, textb, idsb, uwb_v, rsw, accb, denb, isem, wsem):
        wid = lax.axis_index("s") * _NC + lax.axis_index("c")
        pltpu.sync_copy(uwb_hbm, uwb_v)
        pltpu.sync_copy(rs_hbm.at[pl.ds(wid * _L, _L)], rsw)
        rvec = rsw[pl.ds(0, _L)]
        rcut = [rvec[i] for i in range(NPASS + 1)]
        icut = [rvec[NPASS + 1 + i] for i in range(NPASS + 1)]
        zvec = jnp.zeros((_L,), jnp.float32)
        ubv = uwb_v[pl.ds(D, _L)]
        ub = ubv[0]
        uwr = [uwb_v[pl.ds(j * _L, _L)] for j in range(NKD)]
        lane = lax.iota(jnp.int32, _L)
        perms = [lax.rem(lane + (_L >> (s + 1)), _L) for s in range(4)]

        def issue(b0, b, parity):
            bs = b0 + b * RB
            pltpu.async_copy(tid_hbm.at[pl.ds(bs, RB)],
                             idsb.at[pl.ds(parity * RB, RB)],
                             isem.at[parity])
            pltpu.async_copy(rel_hbm.at[pl.ds(bs, RB), :],
                             relb.at[parity], isem.at[parity])
            pltpu.async_copy(text_hbm.at[pl.ds(bs, RB), :],
                             textb.at[parity], isem.at[parity])

        def wait_in(parity):
            pltpu.make_async_copy(tid_hbm.at[pl.ds(0, RB)],
                                  idsb.at[pl.ds(0, RB)],
                                  isem.at[parity]).wait()
            pltpu.make_async_copy(rel_hbm.at[pl.ds(0, RB), :],
                                  relb.at[0], isem.at[parity]).wait()
            pltpu.make_async_copy(text_hbm.at[pl.ds(0, RB), :],
                                  textb.at[0], isem.at[parity]).wait()

        def sel(vals, pi):
            x = vals[-1]
            for i in range(NPASS - 2, -1, -1):
                x = jnp.where(pi == i, vals[i], x)
            return x

        def one_pass(pi, pc):
            rp0 = sel(rcut[:-1], pi)
            rp1 = sel(rcut[1:], pi)
            base = sel(icut[:-1], pi)
            nseg = sel([icut[i + 1] - icut[i] for i in range(NPASS)], pi)

            def zacc(i, c):
                accb[pl.ds(i * _L, _L)] = zvec
                return c
            lax.fori_loop(0, NSEGP * D2 // _L, zacc, 0, unroll=8)
            def zden(i, c):
                denb[pl.ds(i * _L, _L)] = zvec
                return c
            lax.fori_loop(0, NSEGP, zden, 0, unroll=8)

            b0 = (rp0 // RB) * RB
            nblk = jnp.maximum((rp1 - b0 + RB - 1) // RB, 0)

            @pl.when(nblk > 0)
            def _():
                issue(b0, 0, 0)
            @pl.when(nblk > 1)
            def _():
                issue(b0, 1, 1)

            def blk(b, carry):
                parity = lax.rem(b, NBUF)
                bstart = b0 + b * RB
                @pl.when(b + 2 < nblk)
                def _():
                    issue(b0, b + 2, lax.rem(b + 2, NBUF))
                wait_in(parity)

                rp = relb.at[parity]
                tp = textb.at[parity]

                for g in range(NG):
                    idv = idsb[pl.ds(parity * RB + g * _L, _L)]
                    for k in range(_L):
                        row = g * _L + k
                        gj = bstart + row
                        valid = jnp.logical_and(gj >= rp0, gj < rp1)
                        validf = jnp.where(valid, 1.0, 0.0)
                        li = jnp.clip(idv[k] - base, 0, NSEGP - 1)
                        loff = li * D2

                        rv = [rp[row, pl.ds(j * _L, _L)] for j in range(NKD)]
                        p = rv[0] * uwr[0]
                        for j in range(1, NKD):
                            p = p + rv[j] * uwr[j]
                        for s in range(4):
                            p = p + p.at[perms[s]].get(
                                mode="promise_in_bounds")
                        wsp = jnp.exp(p + ub) * validf

                        plsc.addupdate(denb.at[pl.ds(li * _L, _L)], wsp)
                        for j in range(NKD):
                            plsc.addupdate(accb.at[pl.ds(loff + j * _L, _L)],
                                           rv[j] * wsp)
                        for j in range(NKD):
                            tv = tp[row, pl.ds(j * _L, _L)]
                            plsc.addupdate(
                                accb.at[pl.ds(loff + (NKD + j) * _L, _L)],
                                tv * wsp)
                return carry

            lax.fori_loop(0, nblk, blk, 0)

            def wout(li, c):
                dv = 1.0 / (denb[pl.ds(li * _L, _L)] + 1e-16)
                def sc_j(j, c2):
                    off = li * D2 + j * _L
                    accb[pl.ds(off, _L)] = accb[pl.ds(off, _L)] * dv
                    return c2
                lax.fori_loop(0, NK2, sc_j, 0, unroll=8)
                pltpu.async_copy(
                    accb.at[pl.ds(li * D2, D2)],
                    out_hbm.at[pl.ds((base + li) * D2, D2)], wsem)
                return c
            lax.fori_loop(0, nseg, wout, 0)
            def wdrain(i, c):
                pltpu.make_async_copy(out_hbm.at[pl.ds(0, D2)],
                                      accb.at[pl.ds(0, D2)], wsem).wait()
                return c
            lax.fori_loop(0, nseg, wdrain, 0)
            return pc

        lax.fori_loop(0, NPASS, one_pass, 0)

    mesh = plsc.VectorSubcoreMesh(core_axis_name="c", subcore_axis_name="s",
                                  num_cores=_NC, num_subcores=_NS)
    return pl.kernel(
        body,
        out_type=jax.ShapeDtypeStruct((ENT * D2,), jnp.float32),
        mesh=mesh,
        compiler_params=pltpu.CompilerParams(needs_layout_passes=False),
        scratch_types=[
            pltpu.VMEM((NBUF, RB, D), jnp.float32),
            pltpu.VMEM((NBUF, RB, D), jnp.float32),
            pltpu.VMEM((NBUF * RB,), jnp.int32),
            pltpu.VMEM((D + _L,), jnp.float32),
            pltpu.VMEM((_L,), jnp.int32),
            pltpu.VMEM((NSEGP * D2,), jnp.float32),
            pltpu.VMEM((NSEGP * _L,), jnp.float32),
            pltpu.SemaphoreType.DMA((NBUF,)),
            pltpu.SemaphoreType.DMA,
        ],
    )


def kernel(ent_num, Textid, Text, Text_rel, u_w, u_b):
    del ent_num
    E, D = Text.shape
    NPASS = 3
    lo = jnp.array([(t * _ENT) // _NW for t in range(_NW)], dtype=jnp.int32)
    hi = jnp.array([((t + 1) * _ENT) // _NW for t in range(_NW)],
                   dtype=jnp.int32)
    n = hi - lo
    icuts = [lo + (p * n + NPASS - 1) // NPASS for p in range(NPASS + 1)]
    icut = jnp.stack(icuts, axis=1)
    rs = jnp.searchsorted(Textid, icut.reshape(-1)).astype(
        jnp.int32).reshape(_NW, NPASS + 1)
    rsw = jnp.concatenate([rs, icut], axis=1)
    rsw = jnp.pad(rsw, ((0, 0), (0, _L - 2 * (NPASS + 1)))).reshape(-1)
    uwb = jnp.concatenate([u_w.reshape(-1), u_b.reshape(-1),
                           jnp.zeros((_L - 1,), jnp.float32)])
    sc = _build(E, _ENT, D, 32)
    out = sc(Textid, Text_rel, Text, uwb, rsw)
    return out.reshape(_ENT, 2 * D)


# speedup vs baseline: 1.0177x; 1.0177x over previous
import jax
import jax.numpy as jnp
from jax import lax
from jax.experimental import pallas as pl
from jax.experimental.pallas import tpu as pltpu
from jax.experimental.pallas import tpu_sc as plsc

_L = 16
_NC = 2
_NS = 16
_NW = _NC * _NS
_ENT = 10000


def _build(E, ENT, D, RB):
    D2 = 2 * D
    NKD = D // _L
    NK2 = D2 // _L
    NG = RB // _L
    NSEG = (ENT // _NW + 2 + 1) // 2
    NSEGP = ((NSEG + 7) // 8) * 8

    def body(tid_hbm, rel_hbm, text_hbm, uwb_hbm, rs_hbm, out_hbm,
             relb, textb, idsb, uwb_v, rsw, accb, denb, isem, wsem):
        wid = lax.axis_index("s") * _NC + lax.axis_index("c")
        pltpu.sync_copy(uwb_hbm, uwb_v)
        pltpu.sync_copy(rs_hbm.at[pl.ds(wid * _L, _L)], rsw)
        rvec = rsw[pl.ds(0, _L)]
        r0 = rvec[0]
        rmid = rvec[1]
        r1 = rvec[2]
        lo = rvec[3]
        mid = rvec[4]
        hi = rvec[5]
        zvec = jnp.zeros((_L,), jnp.float32)
        ubv = uwb_v[pl.ds(D, _L)]
        ub = ubv[0]
        uwr = [uwb_v[pl.ds(j * _L, _L)] for j in range(NKD)]
        lane = lax.iota(jnp.int32, _L)
        perms = [lax.rem(lane + (_L >> (s + 1)), _L) for s in range(4)]

        def issue(b0, b, parity):
            bs = b0 + b * RB
            pltpu.async_copy(tid_hbm.at[pl.ds(bs, RB)],
                             idsb.at[pl.ds(parity * RB, RB)], isem)
            pltpu.async_copy(rel_hbm.at[pl.ds(bs, RB), :],
                             relb.at[parity], isem)
            pltpu.async_copy(text_hbm.at[pl.ds(bs, RB), :],
                             textb.at[parity], isem)

        def wait_in():
            pltpu.make_async_copy(tid_hbm.at[pl.ds(0, RB)],
                                  idsb.at[pl.ds(0, RB)], isem).wait()
            pltpu.make_async_copy(rel_hbm.at[pl.ds(0, RB), :],
                                  relb.at[0], isem).wait()
            pltpu.make_async_copy(text_hbm.at[pl.ds(0, RB), :],
                                  textb.at[0], isem).wait()

        def one_pass(pi, pc):
            first = pi == 0
            rp0 = jnp.where(first, r0, rmid)
            rp1 = jnp.where(first, rmid, r1)
            base = jnp.where(first, lo, mid)
            nseg = jnp.where(first, mid - lo, hi - mid)

            def zacc(i, c):
                accb[pl.ds(i * _L, _L)] = zvec
                return c
            lax.fori_loop(0, NSEGP * D2 // _L, zacc, 0, unroll=8)
            def zden(i, c):
                denb[pl.ds(i * _L, _L)] = zvec
                return c
            lax.fori_loop(0, NSEGP, zden, 0, unroll=8)

            b0 = (rp0 // RB) * RB
            nblk = jnp.maximum((rp1 - b0 + RB - 1) // RB, 0)

            @pl.when(nblk > 0)
            def _():
                issue(b0, 0, 0)

            def blk(b, carry):
                parity = lax.rem(b, 2)
                bstart = b0 + b * RB
                wait_in()
                @pl.when(b + 1 < nblk)
                def _():
                    issue(b0, b + 1, 1 - parity)

                rp = relb.at[parity]
                tp = textb.at[parity]

                for g in range(NG):
                    idv = idsb[pl.ds(parity * RB + g * _L, _L)]
                    for k in range(_L):
                        row = g * _L + k
                        gj = bstart + row
                        valid = jnp.logical_and(gj >= rp0, gj < rp1)
                        validf = jnp.where(valid, 1.0, 0.0)
                        li = jnp.clip(idv[k] - base, 0, NSEGP - 1)
                        loff = li * D2

                        rv = [rp[row, pl.ds(j * _L, _L)] for j in range(NKD)]
                        p = rv[0] * uwr[0]
                        for j in range(1, NKD):
                            p = p + rv[j] * uwr[j]
                        for s in range(4):
                            p = p + p.at[perms[s]].get(
                                mode="promise_in_bounds")
                        wsp = jnp.exp(p + ub) * validf

                        plsc.addupdate(denb.at[pl.ds(li * _L, _L)], wsp)
                        for j in range(NKD):
                            plsc.addupdate(accb.at[pl.ds(loff + j * _L, _L)],
                                           rv[j] * wsp)
                        for j in range(NKD):
                            tv = tp[row, pl.ds(j * _L, _L)]
                            plsc.addupdate(
                                accb.at[pl.ds(loff + (NKD + j) * _L, _L)],
                                tv * wsp)
                return carry

            lax.fori_loop(0, nblk, blk, 0)

            def wout(li, c):
                dv = 1.0 / (denb[pl.ds(li * _L, _L)] + 1e-16)
                def sc_j(j, c2):
                    off = li * D2 + j * _L
                    accb[pl.ds(off, _L)] = accb[pl.ds(off, _L)] * dv
                    return c2
                lax.fori_loop(0, NK2, sc_j, 0, unroll=8)
                pltpu.async_copy(
                    accb.at[pl.ds(li * D2, D2)],
                    out_hbm.at[pl.ds((base + li) * D2, D2)], wsem)
                return c
            lax.fori_loop(0, nseg, wout, 0)
            def wdrain(i, c):
                pltpu.make_async_copy(out_hbm.at[pl.ds(0, D2)],
                                      accb.at[pl.ds(0, D2)], wsem).wait()
                return c
            lax.fori_loop(0, nseg, wdrain, 0)
            return pc

        lax.fori_loop(0, 2, one_pass, 0)

    mesh = plsc.VectorSubcoreMesh(core_axis_name="c", subcore_axis_name="s",
                                  num_cores=_NC, num_subcores=_NS)
    return pl.kernel(
        body,
        out_type=jax.ShapeDtypeStruct((ENT * D2,), jnp.float32),
        mesh=mesh,
        compiler_params=pltpu.CompilerParams(needs_layout_passes=False),
        scratch_types=[
            pltpu.VMEM((2, RB, D), jnp.float32),
            pltpu.VMEM((2, RB, D), jnp.float32),
            pltpu.VMEM((2 * RB,), jnp.int32),
            pltpu.VMEM((D + _L,), jnp.float32),
            pltpu.VMEM((_L,), jnp.int32),
            pltpu.VMEM((NSEGP * D2,), jnp.float32),
            pltpu.VMEM((NSEGP * _L,), jnp.float32),
            pltpu.SemaphoreType.DMA,
            pltpu.SemaphoreType.DMA,
        ],
    )


def kernel(ent_num, Textid, Text, Text_rel, u_w, u_b):
    del ent_num
    E, D = Text.shape
    lo = jnp.array([(t * _ENT) // _NW for t in range(_NW)], dtype=jnp.int32)
    hi = jnp.array([((t + 1) * _ENT) // _NW for t in range(_NW)],
                   dtype=jnp.int32)
    mid = lo + (hi - lo + 1) // 2
    cuts = jnp.stack([lo, mid, hi], axis=1).reshape(-1)
    rs = jnp.searchsorted(Textid, cuts).astype(jnp.int32).reshape(_NW, 3)
    rsw = jnp.concatenate(
        [rs, lo[:, None], mid[:, None], hi[:, None]], axis=1)
    rsw = jnp.pad(rsw, ((0, 0), (0, _L - 6))).reshape(-1)
    uwb = jnp.concatenate([u_w.reshape(-1), u_b.reshape(-1),
                           jnp.zeros((_L - 1,), jnp.float32)])
    sc = _build(E, _ENT, D, 32)
    out = sc(Textid, Text_rel, Text, uwb, rsw)
    return out.reshape(_ENT, 2 * D)
